# baseline (speedup 1.0000x reference)
import functools

import jax
import jax.numpy as jnp
from jax import lax
from jax.experimental import pallas as pl
from jax.experimental.pallas import tpu as pltpu
from jax.experimental.pallas import tpu_sc as plsc

_NE = 8192
_D = 64
_BM = 576
_SB = 96
_LW = 128
_NC = _NE // _LW
_GROUP = 24
_NG = _BM // _GROUP
_INTERPRET = False

_NW = 32


def _vq_argmin_body(x_ref, xn_ref, emb_ref, en_ref, idx_ref, l_ref):
    x = x_ref[...]
    x2 = x + x
    mm2 = lax.dot_general(x2, emb_ref[...], (((1,), (1,)), ((), ())),
                          preferred_element_type=jnp.float32)
    en = en_ref[...]
    xn = xn_ref[...]
    lane_f = lax.broadcasted_iota(jnp.int32, (_SB, _LW), 1).astype(jnp.float32)
    idx_parts = []
    min_parts = []
    for sb in range(_BM // _SB):
        xn_s = xn[sb * _SB:(sb + 1) * _SB, :]
        acc_v = jnp.full((_SB, _LW), jnp.inf, dtype=jnp.float32)
        acc_c = jnp.zeros((_SB, _LW), dtype=jnp.float32)
        for c in range(_NC):
            en_c = en[:, c * _LW:(c + 1) * _LW]
            mm_c = mm2[sb * _SB:(sb + 1) * _SB, c * _LW:(c + 1) * _LW]
            d_c = (xn_s + en_c) - mm_c
            better = d_c < acc_v
            acc_v = jnp.where(better, d_c, acc_v)
            acc_c = jnp.where(better, float(c), acc_c)
        minval = jnp.min(acc_v, axis=1, keepdims=True)
        cand = jnp.where(acc_v == minval, acc_c * float(_LW) + lane_f,
                         float(_NE))
        idx_f = jnp.min(cand, axis=1, keepdims=True)
        idx_parts.append(idx_f.astype(jnp.int32))
        min_parts.append(minval)
    run_idx = jnp.concatenate(idx_parts, axis=0)
    run_min = jnp.concatenate(min_parts, axis=0)
    idx_ref[...] = run_idx
    gi = lax.broadcasted_iota(jnp.int32, (_NG, _BM), 0)
    ri = lax.broadcasted_iota(jnp.int32, (_NG, _BM), 1)
    gmat = (ri // _GROUP == gi).astype(jnp.float32)
    gsum = lax.dot_general(gmat, run_min, (((1,), (0,)), ((), ())),
                           precision=lax.Precision.HIGHEST,
                           preferred_element_type=jnp.float32)
    l_ref[...] = gsum * ((1.0 + 0.25) / (_GROUP * _D))


def _argmin_call(x_flat, xn, embeddings, en, M):
    return pl.pallas_call(
        _vq_argmin_body,
        grid=(M // _BM,),
        in_specs=[
            pl.BlockSpec((_BM, _D), lambda i: (i, 0)),
            pl.BlockSpec((_BM, 1), lambda i: (i, 0)),
            pl.BlockSpec((_NE, _D), lambda i: (0, 0)),
            pl.BlockSpec((1, _NE), lambda i: (0, 0)),
        ],
        out_specs=[
            pl.BlockSpec((_BM, 1), lambda i: (i, 0)),
            pl.BlockSpec((_NG, 1), lambda i: (i, 0)),
        ],
        out_shape=[
            jax.ShapeDtypeStruct((M, 1), jnp.int32),
            jax.ShapeDtypeStruct((M // _GROUP, 1), jnp.float32),
        ],
        interpret=_INTERPRET,
    )(x_flat, xn, embeddings, en)


def _make_sc_gather(M):
    b_per_w = M // _NW
    mesh = plsc.VectorSubcoreMesh(core_axis_name="c", subcore_axis_name="s")

    @functools.partial(
        pl.kernel,
        mesh=mesh,
        out_type=jax.ShapeDtypeStruct((M, _D), jnp.float32),
        scratch_types=[
            pltpu.VMEM((b_per_w,), jnp.int32),
            pltpu.VMEM((b_per_w, _D), jnp.float32),
            pltpu.SemaphoreType.DMA,
        ],
        compiler_params=pltpu.CompilerParams(use_tc_tiling_on_sc=False),
    )
    def gather_st(emb_hbm, idx_hbm, out_hbm, idx_v, rows_v, sem):
        wid = lax.axis_index("s") * 2 + lax.axis_index("c")
        base = wid * b_per_w
        pltpu.sync_copy(idx_hbm.at[pl.ds(base, b_per_w)], idx_v)
        pltpu.async_copy(emb_hbm.at[idx_v], rows_v, sem).wait()
        pltpu.sync_copy(rows_v, out_hbm.at[pl.ds(base, b_per_w)])

    return gather_st


def kernel(x, embeddings):
    B, H, W, D = x.shape
    M = B * H * W
    x_flat = x.reshape(M, D)
    xn = jnp.sum(x_flat ** 2, axis=1, keepdims=True)
    en = jnp.sum(embeddings ** 2, axis=1).reshape(1, _NE)
    half = M // 2
    sc_gather = _make_sc_gather(half)
    idx1, l1 = _argmin_call(x_flat[:half], xn[:half], embeddings, en, half)
    xq1 = sc_gather(embeddings, idx1.reshape(half))
    idx2, l2 = _argmin_call(x_flat[half:], xn[half:], embeddings, en, half)
    xq2 = sc_gather(embeddings, idx2.reshape(half))
    xq_flat = jnp.concatenate([xq1, xq2], axis=0)
    l_col = jnp.concatenate([l1, l2], axis=0)
    x_q_st = xq_flat.reshape(B, H, W, D)
    l = l_col.reshape(B, H)
    return (x_q_st, l)


# speedup vs baseline: 1.0511x; 1.0511x over previous
import functools

import jax
import jax.numpy as jnp
from jax import lax
from jax.experimental import pallas as pl
from jax.experimental.pallas import tpu as pltpu
from jax.experimental.pallas import tpu_sc as plsc

_NE = 8192
_D = 64
_BM = 1152
_SB = 96
_LW = 128
_NC = _NE // _LW
_GROUP = 24
_NG = _BM // _GROUP
_INTERPRET = False

_NW = 32


def _vq_argmin_body(x_ref, xn_ref, emb_ref, en_ref, idx_ref, l_ref):
    x = x_ref[...]
    x2 = x + x
    mm2 = lax.dot_general(x2, emb_ref[...], (((1,), (1,)), ((), ())),
                          preferred_element_type=jnp.float32)
    en = en_ref[...]
    xn = xn_ref[...]
    lane_f = lax.broadcasted_iota(jnp.int32, (_SB, _LW), 1).astype(jnp.float32)
    idx_parts = []
    min_parts = []
    for sb in range(_BM // _SB):
        xn_s = xn[sb * _SB:(sb + 1) * _SB, :]
        acc_v = jnp.full((_SB, _LW), jnp.inf, dtype=jnp.float32)
        acc_c = jnp.zeros((_SB, _LW), dtype=jnp.float32)
        for c in range(_NC):
            en_c = en[:, c * _LW:(c + 1) * _LW]
            mm_c = mm2[sb * _SB:(sb + 1) * _SB, c * _LW:(c + 1) * _LW]
            d_c = (xn_s + en_c) - mm_c
            better = d_c < acc_v
            acc_v = jnp.where(better, d_c, acc_v)
            acc_c = jnp.where(better, float(c), acc_c)
        minval = jnp.min(acc_v, axis=1, keepdims=True)
        cand = jnp.where(acc_v == minval, acc_c * float(_LW) + lane_f,
                         float(_NE))
        idx_f = jnp.min(cand, axis=1, keepdims=True)
        idx_parts.append(idx_f.astype(jnp.int32))
        min_parts.append(minval)
    run_idx = jnp.concatenate(idx_parts, axis=0)
    run_min = jnp.concatenate(min_parts, axis=0)
    idx_ref[...] = run_idx
    gi = lax.broadcasted_iota(jnp.int32, (_NG, _BM), 0)
    ri = lax.broadcasted_iota(jnp.int32, (_NG, _BM), 1)
    gmat = (ri // _GROUP == gi).astype(jnp.float32)
    gsum = lax.dot_general(gmat, run_min, (((1,), (0,)), ((), ())),
                           precision=lax.Precision.HIGHEST,
                           preferred_element_type=jnp.float32)
    l_ref[...] = gsum * ((1.0 + 0.25) / (_GROUP * _D))


def _argmin_call(x_flat, xn, embeddings, en, M):
    return pl.pallas_call(
        _vq_argmin_body,
        grid=(M // _BM,),
        in_specs=[
            pl.BlockSpec((_BM, _D), lambda i: (i, 0)),
            pl.BlockSpec((_BM, 1), lambda i: (i, 0)),
            pl.BlockSpec((_NE, _D), lambda i: (0, 0)),
            pl.BlockSpec((1, _NE), lambda i: (0, 0)),
        ],
        out_specs=[
            pl.BlockSpec((_BM, 1), lambda i: (i, 0)),
            pl.BlockSpec((_NG, 1), lambda i: (i, 0)),
        ],
        out_shape=[
            jax.ShapeDtypeStruct((M, 1), jnp.int32),
            jax.ShapeDtypeStruct((M // _GROUP, 1), jnp.float32),
        ],
        interpret=_INTERPRET,
    )(x_flat, xn, embeddings, en)


def _make_sc_gather(M):
    b_per_w = M // _NW
    mesh = plsc.VectorSubcoreMesh(core_axis_name="c", subcore_axis_name="s")

    @functools.partial(
        pl.kernel,
        mesh=mesh,
        out_type=jax.ShapeDtypeStruct((M, _D), jnp.float32),
        scratch_types=[
            pltpu.VMEM((b_per_w,), jnp.int32),
            pltpu.VMEM((b_per_w, _D), jnp.float32),
            pltpu.SemaphoreType.DMA,
        ],
        compiler_params=pltpu.CompilerParams(use_tc_tiling_on_sc=False),
    )
    def gather_st(emb_hbm, idx_hbm, out_hbm, idx_v, rows_v, sem):
        wid = lax.axis_index("s") * 2 + lax.axis_index("c")
        base = wid * b_per_w
        pltpu.sync_copy(idx_hbm.at[pl.ds(base, b_per_w)], idx_v)
        pltpu.async_copy(emb_hbm.at[idx_v], rows_v, sem).wait()
        pltpu.sync_copy(rows_v, out_hbm.at[pl.ds(base, b_per_w)])

    return gather_st


def kernel(x, embeddings):
    B, H, W, D = x.shape
    M = B * H * W
    x_flat = x.reshape(M, D)
    xn = jnp.sum(x_flat ** 2, axis=1, keepdims=True)
    en = jnp.sum(embeddings ** 2, axis=1).reshape(1, _NE)
    idx_col, l_col = _argmin_call(x_flat, xn, embeddings, en, M)
    xq_flat = _make_sc_gather(M)(embeddings, idx_col.reshape(M))
    x_q_st = xq_flat.reshape(B, H, W, D)
    l = l_col.reshape(B, H)
    return (x_q_st, l)


# speedup vs baseline: 1.1519x; 1.0959x over previous
import functools

import jax
import jax.numpy as jnp
from jax import lax
from jax.experimental import pallas as pl
from jax.experimental.pallas import tpu as pltpu
from jax.experimental.pallas import tpu_sc as plsc

_NE = 8192
_D = 64
_BM = 576
_SB = 96
_LW = 128
_NC = _NE // _LW
_GROUP = 24
_NG = _BM // _GROUP
_INTERPRET = False

_NW = 32


def _vq_argmin_body(x_ref, xn_ref, emb_ref, en_ref, idx_ref, l_ref):
    x = x_ref[...]
    x2 = x + x
    mm2 = lax.dot_general(x2, emb_ref[...], (((1,), (1,)), ((), ())),
                          preferred_element_type=jnp.float32)
    en = en_ref[...]
    xn = xn_ref[...]
    lane_f = lax.broadcasted_iota(jnp.int32, (_SB, _LW), 1).astype(jnp.float32)
    idx_parts = []
    min_parts = []
    for sb in range(_BM // _SB):
        xn_s = xn[sb * _SB:(sb + 1) * _SB, :]
        acc_v = jnp.full((_SB, _LW), jnp.inf, dtype=jnp.float32)
        acc_c = jnp.zeros((_SB, _LW), dtype=jnp.float32)
        for c in range(_NC):
            en_c = en[:, c * _LW:(c + 1) * _LW]
            mm_c = mm2[sb * _SB:(sb + 1) * _SB, c * _LW:(c + 1) * _LW]
            d_c = (xn_s + en_c) - mm_c
            better = d_c < acc_v
            acc_v = jnp.where(better, d_c, acc_v)
            acc_c = jnp.where(better, float(c), acc_c)
        minval = jnp.min(acc_v, axis=1, keepdims=True)
        cand = jnp.where(acc_v == minval, acc_c * float(_LW) + lane_f,
                         float(_NE))
        idx_f = jnp.min(cand, axis=1, keepdims=True)
        idx_parts.append(idx_f.astype(jnp.int32))
        min_parts.append(minval)
    run_idx = jnp.concatenate(idx_parts, axis=0)
    run_min = jnp.concatenate(min_parts, axis=0)
    idx_ref[...] = run_idx
    gi = lax.broadcasted_iota(jnp.int32, (_NG, _BM), 0)
    ri = lax.broadcasted_iota(jnp.int32, (_NG, _BM), 1)
    gmat = (ri // _GROUP == gi).astype(jnp.float32)
    gsum = lax.dot_general(gmat, run_min, (((1,), (0,)), ((), ())),
                           precision=lax.Precision.HIGHEST,
                           preferred_element_type=jnp.float32)
    l_ref[...] = gsum * ((1.0 + 0.25) / (_GROUP * _D))


def _argmin_call(x_flat, xn, embeddings, en, M):
    return pl.pallas_call(
        _vq_argmin_body,
        grid=(M // _BM,),
        in_specs=[
            pl.BlockSpec((_BM, _D), lambda i: (i, 0)),
            pl.BlockSpec((_BM, 1), lambda i: (i, 0)),
            pl.BlockSpec((_NE, _D), lambda i: (0, 0)),
            pl.BlockSpec((1, _NE), lambda i: (0, 0)),
        ],
        out_specs=[
            pl.BlockSpec((_BM, 1), lambda i: (i, 0)),
            pl.BlockSpec((_NG, 1), lambda i: (i, 0)),
        ],
        out_shape=[
            jax.ShapeDtypeStruct((M, 1), jnp.int32),
            jax.ShapeDtypeStruct((M // _GROUP, 1), jnp.float32),
        ],
        interpret=_INTERPRET,
    )(x_flat, xn, embeddings, en)


def _make_sc_gather(M):
    b_per_w = M // _NW
    mesh = plsc.VectorSubcoreMesh(core_axis_name="c", subcore_axis_name="s")

    @functools.partial(
        pl.kernel,
        mesh=mesh,
        out_type=jax.ShapeDtypeStruct((M, _D), jnp.float32),
        scratch_types=[
            pltpu.VMEM((b_per_w,), jnp.int32),
            pltpu.VMEM((b_per_w, _D), jnp.float32),
            pltpu.SemaphoreType.DMA,
        ],
        compiler_params=pltpu.CompilerParams(use_tc_tiling_on_sc=False),
    )
    def gather_st(emb_hbm, idx_hbm, out_hbm, idx_v, rows_v, sem):
        wid = lax.axis_index("s") * 2 + lax.axis_index("c")
        base = wid * b_per_w
        pltpu.sync_copy(idx_hbm.at[pl.ds(base, b_per_w)], idx_v)
        pltpu.async_copy(emb_hbm.at[idx_v], rows_v, sem).wait()
        pltpu.sync_copy(rows_v, out_hbm.at[pl.ds(base, b_per_w)])

    return gather_st


def kernel(x, embeddings):
    B, H, W, D = x.shape
    M = B * H * W
    x_flat = x.reshape(M, D)
    xn = jnp.sum(x_flat ** 2, axis=1, keepdims=True)
    en = jnp.sum(embeddings ** 2, axis=1).reshape(1, _NE)
    idx_col, l_col = _argmin_call(x_flat, xn, embeddings, en, M)
    xq_flat = _make_sc_gather(M)(embeddings, idx_col.reshape(M))
    x_q_st = xq_flat.reshape(B, H, W, D)
    l = l_col.reshape(B, H)
    return (x_q_st, l)


# speedup vs baseline: 1.1530x; 1.0010x over previous
import functools

import jax
import jax.numpy as jnp
from jax import lax
from jax.experimental import pallas as pl
from jax.experimental.pallas import tpu as pltpu
from jax.experimental.pallas import tpu_sc as plsc

_NE = 8192
_D = 64
_BM = 576
_SB = 96
_LW = 128
_NC = _NE // _LW
_GROUP = 24
_NG = _BM // _GROUP

_NW = 32


def _vq_argmin_body(x_ref, xn_ref, emb_ref, en_ref, idx_ref, l_ref):
    x = x_ref[...]
    x2 = x + x
    mm2 = lax.dot_general(x2, emb_ref[...], (((1,), (1,)), ((), ())),
                          preferred_element_type=jnp.float32)
    en = en_ref[...]
    xn = xn_ref[...]
    lane_f = lax.broadcasted_iota(jnp.int32, (_SB, _LW), 1).astype(jnp.float32)
    idx_parts = []
    min_parts = []
    for sb in range(_BM // _SB):
        xn_s = xn[sb * _SB:(sb + 1) * _SB, :]
        acc_v = jnp.full((_SB, _LW), jnp.inf, dtype=jnp.float32)
        acc_c = jnp.zeros((_SB, _LW), dtype=jnp.float32)
        for c in range(_NC):
            en_c = en[:, c * _LW:(c + 1) * _LW]
            mm_c = mm2[sb * _SB:(sb + 1) * _SB, c * _LW:(c + 1) * _LW]
            d_c = (xn_s + en_c) - mm_c
            better = d_c < acc_v
            acc_v = jnp.where(better, d_c, acc_v)
            acc_c = jnp.where(better, float(c), acc_c)
        minval = jnp.min(acc_v, axis=1, keepdims=True)
        cand = jnp.where(acc_v == minval, acc_c * float(_LW) + lane_f,
                         float(_NE))
        idx_f = jnp.min(cand, axis=1, keepdims=True)
        idx_parts.append(idx_f.astype(jnp.int32))
        min_parts.append(minval)
    run_idx = jnp.concatenate(idx_parts, axis=0)
    run_min = jnp.concatenate(min_parts, axis=0)
    idx_ref[...] = run_idx
    gi = lax.broadcasted_iota(jnp.int32, (_NG, _BM), 0)
    ri = lax.broadcasted_iota(jnp.int32, (_NG, _BM), 1)
    gmat = (ri // _GROUP == gi).astype(jnp.float32)
    gsum = lax.dot_general(gmat, run_min, (((1,), (0,)), ((), ())),
                           precision=lax.Precision.HIGHEST,
                           preferred_element_type=jnp.float32)
    l_ref[...] = gsum * ((1.0 + 0.25) / (_GROUP * _D))


def _argmin_call(x_flat, xn, embeddings, en, M):
    return pl.pallas_call(
        _vq_argmin_body,
        grid=(M // _BM,),
        in_specs=[
            pl.BlockSpec((_BM, _D), lambda i: (i, 0)),
            pl.BlockSpec((_BM, 1), lambda i: (i, 0)),
            pl.BlockSpec((_NE, _D), lambda i: (0, 0)),
            pl.BlockSpec((1, _NE), lambda i: (0, 0)),
        ],
        out_specs=[
            pl.BlockSpec((_BM, 1), lambda i: (i, 0)),
            pl.BlockSpec((_NG, 1), lambda i: (i, 0)),
        ],
        out_shape=[
            jax.ShapeDtypeStruct((M, 1), jnp.int32),
            jax.ShapeDtypeStruct((M // _GROUP, 1), jnp.float32),
        ],
    )(x_flat, xn, embeddings, en)


def _make_sc_gather(M):
    b_per_w = M // _NW
    mesh = plsc.VectorSubcoreMesh(core_axis_name="c", subcore_axis_name="s")

    @functools.partial(
        pl.kernel,
        mesh=mesh,
        out_type=jax.ShapeDtypeStruct((M, _D), jnp.float32),
        scratch_types=[
            pltpu.VMEM((b_per_w,), jnp.int32),
            pltpu.VMEM((b_per_w, _D), jnp.float32),
            pltpu.SemaphoreType.DMA,
        ],
        compiler_params=pltpu.CompilerParams(use_tc_tiling_on_sc=False),
    )
    def gather_st(emb_hbm, idx_hbm, out_hbm, idx_v, rows_v, sem):
        wid = lax.axis_index("s") * 2 + lax.axis_index("c")
        base = wid * b_per_w
        pltpu.sync_copy(idx_hbm.at[pl.ds(base, b_per_w)], idx_v)
        pltpu.async_copy(emb_hbm.at[idx_v], rows_v, sem).wait()
        pltpu.sync_copy(rows_v, out_hbm.at[pl.ds(base, b_per_w)])

    return gather_st


def kernel(x, embeddings):
    B, H, W, D = x.shape
    M = B * H * W
    x_flat = x.reshape(M, D)
    xn = jnp.sum(x_flat ** 2, axis=1, keepdims=True)
    en = jnp.sum(embeddings ** 2, axis=1).reshape(1, _NE)
    idx_col, l_col = _argmin_call(x_flat, xn, embeddings, en, M)
    xq_flat = _make_sc_gather(M)(embeddings, idx_col.reshape(M))
    x_q_st = xq_flat.reshape(B, H, W, D)
    l = l_col.reshape(B, H)
    return (x_q_st, l)
